# Initial kernel scaffold; baseline (speedup 1.0000x reference)
import jax
import jax.numpy as jnp
from jax.experimental import pallas as pl


def kernel(input, table):
    raise NotImplementedError("write your pallas kernel here")


# speedup vs baseline: 197.3678x; 197.3678x over previous
import functools

import jax
import jax.numpy as jnp
from jax import lax
from jax.experimental import pallas as pl
from jax.experimental.pallas import tpu as pltpu
from jax.experimental.pallas import tpu_sc as plsc

_LANES = 16


def kernel(input, table):
    B, L = input.shape
    V = table.shape[0]
    N = B * L
    info = plsc.get_sparse_core_info()
    nw = info.num_cores * info.num_subcores
    per_w = N // nw
    assert per_w * nw == N and per_w % _LANES == 0

    flat_idx = input.reshape(N)
    flat_tab = table.reshape(V)

    mesh = plsc.VectorSubcoreMesh(core_axis_name="c", subcore_axis_name="s")

    @functools.partial(
        pl.kernel,
        mesh=mesh,
        compiler_params=pltpu.CompilerParams(needs_layout_passes=False),
        out_type=jax.ShapeDtypeStruct((N,), jnp.float32),
        scratch_types=[
            pltpu.VMEM((per_w,), jnp.int32),
            pltpu.VMEM((per_w,), jnp.float32),
            pltpu.VMEM((V,), jnp.float32),
        ],
    )
    def lookup(idx_hbm, tab_hbm, out_hbm, idx_v, out_v, tab_v):
        wid = lax.axis_index("s") * info.num_cores + lax.axis_index("c")
        base = wid * per_w
        pltpu.sync_copy(tab_hbm, tab_v)
        pltpu.sync_copy(idx_hbm.at[pl.ds(base, per_w)], idx_v)

        def body(i, carry):
            off = pl.multiple_of(i * _LANES, _LANES)
            iv = idx_v[pl.ds(off, _LANES)]
            out_v[pl.ds(off, _LANES)] = plsc.load_gather(tab_v, [iv])
            return carry

        lax.fori_loop(0, per_w // _LANES, body, 0)
        pltpu.sync_copy(out_v, out_hbm.at[pl.ds(base, per_w)])

    return lookup(flat_idx, flat_tab).reshape(B, L)


# speedup vs baseline: 231.2716x; 1.1718x over previous
import functools

import jax
import jax.numpy as jnp
from jax import lax
from jax.experimental import pallas as pl
from jax.experimental.pallas import tpu as pltpu
from jax.experimental.pallas import tpu_sc as plsc

_LANES = 16


def kernel(input, table):
    B, L = input.shape
    V = table.shape[0]
    N = B * L
    info = plsc.get_sparse_core_info()
    nw = info.num_cores * info.num_subcores
    per_w = N // nw
    assert per_w * nw == N and per_w % _LANES == 0

    flat_idx = input.reshape(N)
    flat_tab = table.reshape(V)

    mesh = plsc.VectorSubcoreMesh(core_axis_name="c", subcore_axis_name="s")

    @functools.partial(
        pl.kernel,
        mesh=mesh,
        compiler_params=pltpu.CompilerParams(needs_layout_passes=False),
        out_type=jax.ShapeDtypeStruct((N,), jnp.float32),
        scratch_types=[
            pltpu.VMEM((per_w,), jnp.int32),
            pltpu.VMEM((per_w,), jnp.float32),
            pltpu.VMEM((V,), jnp.float32),
        ],
    )
    def lookup(idx_hbm, tab_hbm, out_hbm, idx_v, out_v, tab_v):
        wid = lax.axis_index("s") * info.num_cores + lax.axis_index("c")
        base = wid * per_w
        pltpu.sync_copy(tab_hbm, tab_v)
        pltpu.sync_copy(idx_hbm.at[pl.ds(base, per_w)], idx_v)

        @plsc.parallel_loop(0, per_w, step=_LANES, unroll=8)
        def body(off):
            iv = idx_v[pl.ds(off, _LANES)]
            out_v[pl.ds(off, _LANES)] = plsc.load_gather(tab_v, [iv])
        pltpu.sync_copy(out_v, out_hbm.at[pl.ds(base, per_w)])

    return lookup(flat_idx, flat_tab).reshape(B, L)


# speedup vs baseline: 238.4283x; 1.0309x over previous
import functools

import jax
import jax.numpy as jnp
from jax import lax
from jax.experimental import pallas as pl
from jax.experimental.pallas import tpu as pltpu
from jax.experimental.pallas import tpu_sc as plsc

_LANES = 16


def kernel(input, table):
    B, L = input.shape
    V = table.shape[0]
    N = B * L
    info = plsc.get_sparse_core_info()
    nw = info.num_cores * info.num_subcores
    per_w = N // nw
    assert per_w * nw == N and per_w % _LANES == 0

    flat_idx = input.reshape(N)
    flat_tab = table.reshape(V)

    mesh = plsc.VectorSubcoreMesh(core_axis_name="c", subcore_axis_name="s")

    @functools.partial(
        pl.kernel,
        mesh=mesh,
        compiler_params=pltpu.CompilerParams(needs_layout_passes=False),
        out_type=jax.ShapeDtypeStruct((N,), jnp.float32),
        scratch_types=[
            pltpu.VMEM((per_w,), jnp.int32),
            pltpu.VMEM((per_w,), jnp.float32),
            pltpu.VMEM((V,), jnp.float32),
            pltpu.SemaphoreType.DMA,
            pltpu.SemaphoreType.DMA,
            pltpu.SemaphoreType.DMA,
            pltpu.SemaphoreType.DMA,
        ],
    )
    def lookup(idx_hbm, tab_hbm, out_hbm, idx_v, out_v, tab_v, si0, si1, so0, so1):
        wid = lax.axis_index("s") * info.num_cores + lax.axis_index("c")
        base = wid * per_w
        half = per_w // 2
        in0 = pltpu.async_copy(
            idx_hbm.at[pl.ds(base, half)], idx_v.at[pl.ds(0, half)], si0)
        in1 = pltpu.async_copy(
            idx_hbm.at[pl.ds(base + half, half)], idx_v.at[pl.ds(half, half)], si1)
        pltpu.sync_copy(tab_hbm, tab_v)
        in0.wait()

        @plsc.parallel_loop(0, half, step=_LANES, unroll=8)
        def body0(off):
            iv = idx_v[pl.ds(off, _LANES)]
            out_v[pl.ds(off, _LANES)] = plsc.load_gather(tab_v, [iv])

        out0 = pltpu.async_copy(
            out_v.at[pl.ds(0, half)], out_hbm.at[pl.ds(base, half)], so0)
        in1.wait()

        @plsc.parallel_loop(half, per_w, step=_LANES, unroll=8)
        def body1(off):
            iv = idx_v[pl.ds(off, _LANES)]
            out_v[pl.ds(off, _LANES)] = plsc.load_gather(tab_v, [iv])

        out1 = pltpu.async_copy(
            out_v.at[pl.ds(half, half)], out_hbm.at[pl.ds(base + half, half)], so1)
        out0.wait()
        out1.wait()

    return lookup(flat_idx, flat_tab).reshape(B, L)
